# baseline (speedup 1.0000x reference)
import functools

import jax
import jax.numpy as jnp
from jax import lax
from jax.experimental import pallas as pl
from jax.experimental.pallas import tpu as pltpu
from jax.experimental.pallas import tpu_sc as plsc

B = 4096 * 26
NW = 32
BPW = B // NW
NJ = 26
ROW = 448

_MESH = plsc.VectorSubcoreMesh(core_axis_name="c", subcore_axis_name="s")


@functools.partial(
    pl.kernel,
    mesh=_MESH,
    out_type=jax.ShapeDtypeStruct((NJ, ROW, 4096), jnp.float32),
    scratch_types=[
        pltpu.VMEM((NJ, 128), jnp.int32),
        pltpu.VMEM((ROW * 16,), jnp.float32),
        pltpu.VMEM((2, ROW, 128), jnp.float32),
        pltpu.SemaphoreType.DMA,
        pltpu.SemaphoreType.DMA,
    ],
    compiler_params=pltpu.CompilerParams(use_tc_tiling_on_sc=True, needs_layout_passes=False),
)
def _emb_sc(z_hbm, tabt_hbm, out_hbm, zblk_v, tabt_v, slab_v, ss0, ss1):
    wid = lax.axis_index("s") * 2 + lax.axis_index("c")
    i0 = wid * 128
    pltpu.sync_copy(tabt_hbm, tabt_v)
    pltpu.sync_copy(z_hbm.at[:, pl.ds(i0, 128)], zblk_v)

    ssem = (ss0, ss1)
    handles = [None, None]

    for j in range(NJ):
        b = j % 2
        if handles[b] is not None:
            handles[b].wait()
        zz = [zblk_v[j, pl.ds(16 * g, 16)] for g in range(8)]

        @plsc.parallel_loop(0, ROW, unroll=8)
        def _body(c, zz=zz, b=b):
            tab16 = tabt_v[pl.ds(c * 16, 16)]
            for g in range(8):
                v = lax.gather(
                    tab16,
                    zz[g][:, None],
                    lax.GatherDimensionNumbers(
                        offset_dims=(),
                        collapsed_slice_dims=(0,),
                        start_index_map=(0,),
                    ),
                    (1,),
                    mode=lax.GatherScatterMode.PROMISE_IN_BOUNDS,
                )
                slab_v[b, c, pl.ds(16 * g, 16)] = v
        handles[b] = pltpu.async_copy(
            slab_v.at[b], out_hbm.at[j, :, pl.ds(i0, 128)], ssem[b]
        )
    handles[0].wait()
    handles[1].wait()


def kernel(inputs, embeddings):
    idx = inputs.T
    tab_t = jnp.pad(embeddings.reshape(14, ROW).T, ((0, 0), (0, 2))).reshape(-1)
    out_t = _emb_sc(idx, tab_t)
    return out_t.reshape(NJ, 14, 32, 4096).transpose(3, 0, 1, 2)


# speedup vs baseline: 1.0289x; 1.0289x over previous
import functools

import jax
import jax.numpy as jnp
from jax import lax
from jax.experimental import pallas as pl
from jax.experimental.pallas import tpu as pltpu
from jax.experimental.pallas import tpu_sc as plsc

B = 4096 * 26
NW = 32
BPW = B // NW
NJ = 26
ROW = 448

_MESH = plsc.VectorSubcoreMesh(core_axis_name="c", subcore_axis_name="s")


@functools.partial(
    pl.kernel,
    mesh=_MESH,
    out_type=jax.ShapeDtypeStruct((NJ, ROW, 4096), jnp.float32),
    scratch_types=[
        pltpu.VMEM((BPW,), jnp.int32),
        pltpu.VMEM((ROW * 16,), jnp.float32),
        pltpu.VMEM((2, ROW, 128), jnp.float32),
        pltpu.SemaphoreType.DMA,
        pltpu.SemaphoreType.DMA,
    ],
    compiler_params=pltpu.CompilerParams(use_tc_tiling_on_sc=True, needs_layout_passes=False),
)
def _emb_sc(z_hbm, tabt_hbm, out_hbm, zblk_v, tabt_v, slab_v, ss0, ss1):
    wid = lax.axis_index("s") * 2 + lax.axis_index("c")
    i0 = wid * 128
    pltpu.sync_copy(tabt_hbm, tabt_v)
    pltpu.sync_copy(z_hbm.at[pl.ds(wid * BPW, BPW)], zblk_v)

    lane26 = lax.iota(jnp.int32, 16) * NJ
    ssem = (ss0, ss1)
    handles = [None, None]

    for j in range(NJ):
        b = j % 2
        if handles[b] is not None:
            handles[b].wait()
        zz = [
            plsc.load_gather(zblk_v, [lane26 + (16 * g * NJ + j)])
            for g in range(8)
        ]

        @plsc.parallel_loop(0, ROW, unroll=4)
        def _body(c, zz=zz, b=b):
            tab16 = tabt_v[pl.ds(c * 16, 16)]
            for g in range(8):
                v = lax.gather(
                    tab16,
                    zz[g][:, None],
                    lax.GatherDimensionNumbers(
                        offset_dims=(),
                        collapsed_slice_dims=(0,),
                        start_index_map=(0,),
                    ),
                    (1,),
                    mode=lax.GatherScatterMode.PROMISE_IN_BOUNDS,
                )
                slab_v[b, c, pl.ds(16 * g, 16)] = v
        handles[b] = pltpu.async_copy(
            slab_v.at[b], out_hbm.at[j, :, pl.ds(i0, 128)], ssem[b]
        )
    handles[0].wait()
    handles[1].wait()


def kernel(inputs, embeddings):
    idx = inputs.reshape(-1)
    tab_t = jnp.pad(embeddings.reshape(14, ROW).T, ((0, 0), (0, 2))).reshape(-1)
    out_t = _emb_sc(idx, tab_t)
    return out_t.reshape(NJ, 14, 32, 4096).transpose(3, 0, 1, 2)


# speedup vs baseline: 1.0455x; 1.0161x over previous
import functools

import jax
import jax.numpy as jnp
from jax import lax
from jax.experimental import pallas as pl
from jax.experimental.pallas import tpu as pltpu
from jax.experimental.pallas import tpu_sc as plsc

B = 4096 * 26
NW = 32
BPW = B // NW
NJ = 26
ROW = 448

_MESH = plsc.VectorSubcoreMesh(core_axis_name="c", subcore_axis_name="s")


@functools.partial(
    pl.kernel,
    mesh=_MESH,
    out_type=jax.ShapeDtypeStruct((NJ, ROW, 4096), jnp.float32),
    scratch_types=[
        pltpu.VMEM((BPW,), jnp.int32),
        pltpu.VMEM((ROW * 16,), jnp.float32),
        pltpu.VMEM((2, ROW, 128), jnp.float32),
        pltpu.SemaphoreType.DMA,
        pltpu.SemaphoreType.DMA,
    ],
    compiler_params=pltpu.CompilerParams(use_tc_tiling_on_sc=True, needs_layout_passes=False),
)
def _emb_sc(z_hbm, tabt_hbm, out_hbm, zblk_v, tabt_v, slab_v, ss0, ss1):
    wid = lax.axis_index("s") * 2 + lax.axis_index("c")
    i0 = wid * 128
    pltpu.sync_copy(tabt_hbm, tabt_v)
    pltpu.sync_copy(z_hbm.at[pl.ds(wid * BPW, BPW)], zblk_v)

    lane26 = lax.iota(jnp.int32, 16) * NJ
    ssem = (ss0, ss1)
    handles = [None, None]

    for j in range(NJ):
        b = j % 2
        if handles[b] is not None:
            handles[b].wait()
        zz = [
            plsc.load_gather(zblk_v, [lane26 + (16 * g * NJ + j)])
            for g in range(8)
        ]

        @plsc.parallel_loop(0, ROW, unroll=2)
        def _body(c, zz=zz, b=b):
            tab16 = tabt_v[pl.ds(c * 16, 16)]
            for g in range(8):
                v = lax.gather(
                    tab16,
                    zz[g][:, None],
                    lax.GatherDimensionNumbers(
                        offset_dims=(),
                        collapsed_slice_dims=(0,),
                        start_index_map=(0,),
                    ),
                    (1,),
                    mode=lax.GatherScatterMode.PROMISE_IN_BOUNDS,
                )
                slab_v[b, c, pl.ds(16 * g, 16)] = v
        handles[b] = pltpu.async_copy(
            slab_v.at[b], out_hbm.at[j, :, pl.ds(i0, 128)], ssem[b]
        )
    handles[0].wait()
    handles[1].wait()


def kernel(inputs, embeddings):
    idx = inputs.reshape(-1)
    tab_t = jnp.pad(embeddings.reshape(14, ROW).T, ((0, 0), (0, 2))).reshape(-1)
    out_t = _emb_sc(idx, tab_t)
    return out_t.reshape(NJ, 14, 32, 4096).transpose(3, 0, 1, 2)
